# baseline (speedup 1.0000x reference)
import functools

import jax
import jax.numpy as jnp
from jax import lax
from jax.experimental import pallas as pl
from jax.experimental.pallas import tpu as pltpu
from jax.experimental.pallas import tpu_sc as plsc

_NC = 2
_NS = 16
_NW = _NC * _NS

_B, _C, _R, _K = 4, 192, 512, 15
_NCH = 8
_N = 8
_BC = _B * _C
_PER_W = _BC // _NW
_ROW_IN = _R * _K
_ROW_OUT = _R * _N
_T_STEPS = _ROW_OUT // 16

_OUT_TYPE = jax.ShapeDtypeStruct((_BC * _NCH, _ROW_OUT), jnp.float32)
_SCRATCH = [
    pltpu.VMEM((_NCH * 16,), jnp.float32),
    pltpu.VMEM((16,), jnp.int32),
    pltpu.VMEM((16,), jnp.float32),
    pltpu.VMEM((_NCH * 16,), jnp.int32),
    pltpu.VMEM((_NCH * 16,), jnp.float32),
    pltpu.VMEM((_ROW_IN,), jnp.float32),
    pltpu.VMEM((_NCH, _ROW_OUT), jnp.float32),
    pltpu.SemaphoreType.DMA,
]


def _body(x_hbm, pw_hbm, out_hbm, pwbuf, ktbl, wtbl, bidx, wvecs, xbuf, obuf, sem):
    iota = lax.iota(jnp.int32, 16)

    pltpu.sync_copy(pw_hbm, pwbuf)
    for i in range(_NCH):
        v = pwbuf[pl.ds(16 * i, 16)]
        rank = jnp.zeros((16,), jnp.int32)
        for j in range(_K):
            vj = plsc.load_gather(pwbuf, [jnp.full((16,), 16 * i + j, jnp.int32)])
            take = (vj < v) | ((vj == v) & (iota > j))
            rank = rank + jnp.where(take, 1, 0)
        sel = rank < _N
        plsc.store_scatter(ktbl, [rank], iota, mask=sel)
        plsc.store_scatter(wtbl, [rank], v, mask=sel)
        lo = iota & 7
        km = plsc.load_gather(ktbl, [lo])
        wm = plsc.load_gather(wtbl, [lo])
        bidx[pl.ds(16 * i, 16)] = km + jnp.where(iota >= 8, _K, 0)
        wvecs[pl.ds(16 * i, 16)] = wm

    wid = lax.axis_index("s") * _NC + lax.axis_index("c")

    def bc_body(t, _):
        bc = wid * _PER_W + t
        b = bc // _C
        c = bc - b * _C
        pltpu.sync_copy(x_hbm.at[bc], xbuf)
        copies = []
        for i in range(_NCH):
            base = bidx[pl.ds(16 * i, 16)]
            wv = wvecs[pl.ds(16 * i, 16)]

            def g_body(g, idxv, i=i, wv=wv):
                obuf[i, pl.ds(g * 16, 16)] = plsc.load_gather(xbuf, [idxv]) * wv
                return idxv + 2 * _K

            lax.fori_loop(0, _T_STEPS, g_body, base, unroll=4)
            row = b * (_NCH * _C) + i * _C + c
            copies.append(pltpu.async_copy(obuf.at[i], out_hbm.at[row], sem))
        for cp in copies:
            cp.wait()
        return 0

    lax.fori_loop(0, _PER_W, bc_body, 0)


@functools.cache
def _call():
    mesh = plsc.VectorSubcoreMesh(
        core_axis_name="c", subcore_axis_name="s", num_cores=_NC, num_subcores=_NS
    )
    return pl.kernel(
        _body,
        out_type=_OUT_TYPE,
        mesh=mesh,
        scratch_types=_SCRATCH,
        compiler_params=pltpu.CompilerParams(needs_layout_passes=False),
    )


def kernel(X, partition_weights):
    pw_sig = jax.nn.sigmoid(partition_weights)
    pw_pad = jnp.concatenate(
        [pw_sig, jnp.full((_NCH, 1), 2.0, jnp.float32)], axis=1
    ).reshape(-1)
    x2 = X.reshape(_BC, _ROW_IN)
    out = _call()(x2, pw_pad)
    return out.reshape(_B, _NCH * _C, _R, _N)
